# baseline (speedup 1.0000x reference)
import functools

import jax
import jax.numpy as jnp
from jax import lax
from jax.experimental import pallas as pl
from jax.experimental.pallas import tpu as pltpu
from jax.experimental.pallas import tpu_sc as plsc

_NC = 2
_NS = 16
_NW = _NC * _NS

_BATCH = 16384
_N_FIELDS = 26
_DIM = 32
_TOTAL = _BATCH * _N_FIELDS
_BLK = 128
_CBLK = _BATCH // _BLK
_SB = 4
_SB_ROWS = _SB * _BLK
_NSB = _TOTAL // (_SB_ROWS * _NW)
_IDX_PER_W = _NSB * _SB_ROWS
_SB_PER_F = _CBLK // _SB


def _make_gather():
    mesh = plsc.VectorSubcoreMesh(core_axis_name="c", subcore_axis_name="s")

    @functools.partial(
        pl.kernel,
        mesh=mesh,
        compiler_params=pltpu.CompilerParams(use_tc_tiling_on_sc=False,
                                             needs_layout_passes=False),
        out_type=jax.ShapeDtypeStruct((_N_FIELDS, 4, _CBLK, 8, 128),
                                      jnp.float32),
        scratch_types=[
            pltpu.VMEM((_IDX_PER_W,), jnp.int32),
            pltpu.VMEM((_SB_ROWS, _DIM), jnp.float32),
            pltpu.VMEM((_SB_ROWS, _DIM), jnp.float32),
            pltpu.VMEM((4, _SB, 8, 128), jnp.float32),
            pltpu.VMEM((4, _SB, 8, 128), jnp.float32),
            pltpu.SemaphoreType.DMA,
            pltpu.SemaphoreType.DMA,
            pltpu.SemaphoreType.DMA,
            pltpu.SemaphoreType.DMA,
        ],
    )
    def gather_kernel(idx_hbm, table_hbm, out_hbm,
                      idx_v, rows0, rows1, tiles0, tiles1, g0, g1, w0, w1):
        wid = lax.axis_index("s") * _NC + lax.axis_index("c")
        base_sb = wid * _NSB
        pltpu.sync_copy(idx_hbm.at[pl.ds(wid * _IDX_PER_W, _IDX_PER_W)],
                        idx_v)

        rows = (rows0, rows1)
        tiles = (tiles0, tiles1)
        gsem = (g0, g1)
        wsem = (w0, w1)
        lane = lax.broadcasted_iota(jnp.int32, (16,), 0)

        def gather_copy(jsb, b):
            return pltpu.make_async_copy(
                table_hbm.at[idx_v.at[pl.ds(jsb * _SB_ROWS, _SB_ROWS)]],
                rows[b], gsem[b])

        def write_copy(gsb, b, r):
            f = gsb // _SB_PER_F
            c0 = lax.rem(gsb, _SB_PER_F) * _SB
            return pltpu.make_async_copy(
                tiles[b].at[r], out_hbm.at[f, r, pl.ds(c0, _SB)], wsem[b])

        for b in (0, 1):
            gather_copy(b, b).start()

        def body(i, carry):
            for b in (0, 1):
                jsb = 2 * i + b
                gsb = base_sb + jsb

                @pl.when(i >= 1)
                def _wait_writes():
                    for r in range(4):
                        write_copy(gsb, b, r).wait()

                gather_copy(jsb, b).wait()

                rows_b = rows[b]
                tiles_b = tiles[b]

                @plsc.parallel_loop(0, 32 * _SB * 8, unroll=8)
                def _transpose(it):
                    j = it // (_SB * 8)
                    cc = lax.rem(it // 8, _SB)
                    e = lax.rem(it, 8)
                    col = jnp.full((16,), j, jnp.int32)
                    vec = plsc.load_gather(
                        rows_b, [128 * cc + 16 * e + lane, col])
                    tiles_b[j // 8, cc, lax.rem(j, 8), pl.ds(16 * e, 16)] = vec

                for r in range(4):
                    write_copy(gsb, b, r).start()

                @pl.when(i < (_NSB // 2) - 1)
                def _next_gather():
                    gather_copy(jsb + 2, b).start()
            return carry

        lax.fori_loop(0, _NSB // 2, body, 0)

        for b in (0, 1):
            gsb = base_sb + _NSB - 2 + b
            for r in range(4):
                write_copy(gsb, b, r).wait()

    return gather_kernel


_gather = _make_gather()


def kernel(x, table):
    idx = x.T.reshape(_TOTAL)
    m = _gather(idx, table)
    return m.transpose(2, 4, 0, 1, 3).reshape(_BATCH, _N_FIELDS, _DIM)


# speedup vs baseline: 1.1143x; 1.1143x over previous
import functools

import jax
import jax.numpy as jnp
from jax import lax
from jax.experimental import pallas as pl
from jax.experimental.pallas import tpu as pltpu
from jax.experimental.pallas import tpu_sc as plsc

_NC = 2
_NS = 16
_NW = _NC * _NS

_BATCH = 16384
_N_FIELDS = 26
_DIM = 32
_TOTAL = _BATCH * _N_FIELDS
_BLK = 128
_CBLK = _BATCH // _BLK
_SB = 4
_SB_ROWS = _SB * _BLK
_NSB = _TOTAL // (_SB_ROWS * _NW)
_IDX_PER_W = _NSB * _SB_ROWS
_SB_PER_F = _CBLK // _SB


def _make_gather():
    mesh = plsc.VectorSubcoreMesh(core_axis_name="c", subcore_axis_name="s")

    @functools.partial(
        pl.kernel,
        mesh=mesh,
        compiler_params=pltpu.CompilerParams(use_tc_tiling_on_sc=False,
                                             needs_layout_passes=False),
        out_type=jax.ShapeDtypeStruct((_N_FIELDS, 4, _CBLK, 8, 128),
                                      jnp.float32),
        scratch_types=[
            pltpu.VMEM((_IDX_PER_W,), jnp.int32),
            pltpu.VMEM((_SB_ROWS, _DIM), jnp.float32),
            pltpu.VMEM((_SB_ROWS, _DIM), jnp.float32),
            pltpu.VMEM((4, _SB, 8, 128), jnp.float32),
            pltpu.VMEM((4, _SB, 8, 128), jnp.float32),
            pltpu.SemaphoreType.DMA,
            pltpu.SemaphoreType.DMA,
            pltpu.SemaphoreType.DMA,
            pltpu.SemaphoreType.DMA,
        ],
    )
    def gather_kernel(idx_hbm, table_hbm, out_hbm,
                      idx_v, rows0, rows1, tiles0, tiles1, g0, g1, w0, w1):
        wid = lax.axis_index("s") * _NC + lax.axis_index("c")
        base_sb = wid * _NSB
        pltpu.sync_copy(idx_hbm.at[pl.ds(wid * _IDX_PER_W, _IDX_PER_W)],
                        idx_v)

        rows = (rows0, rows1)
        tiles = (tiles0, tiles1)
        gsem = (g0, g1)
        wsem = (w0, w1)
        lane = lax.broadcasted_iota(jnp.int32, (16,), 0)
        rvec = [[128 * cc + 16 * e + lane for e in range(8)]
                for cc in range(_SB)]

        def gather_copy(jsb, b):
            return pltpu.make_async_copy(
                table_hbm.at[idx_v.at[pl.ds(jsb * _SB_ROWS, _SB_ROWS)]],
                rows[b], gsem[b])

        def write_copy(gsb, b, r):
            f = gsb // _SB_PER_F
            c0 = lax.rem(gsb, _SB_PER_F) * _SB
            return pltpu.make_async_copy(
                tiles[b].at[r], out_hbm.at[f, r, pl.ds(c0, _SB)], wsem[b])

        for b in (0, 1):
            gather_copy(b, b).start()

        def body(i, carry):
            for b in (0, 1):
                jsb = 2 * i + b
                gsb = base_sb + jsb

                @pl.when(i >= 1)
                def _wait_writes():
                    for r in range(4):
                        write_copy(gsb, b, r).wait()

                gather_copy(jsb, b).wait()

                rows_b = rows[b]
                tiles_b = tiles[b]

                @plsc.parallel_loop(0, 32, unroll=2)
                def _transpose(j):
                    col = jnp.full((16,), j, jnp.int32)
                    r = j // 8
                    s = lax.rem(j, 8)
                    for cc in range(_SB):
                        for e in range(8):
                            vec = plsc.load_gather(rows_b, [rvec[cc][e], col])
                            tiles_b[r, cc, s, pl.ds(16 * e, 16)] = vec

                for r in range(4):
                    write_copy(gsb, b, r).start()

                @pl.when(i < (_NSB // 2) - 1)
                def _next_gather():
                    gather_copy(jsb + 2, b).start()
            return carry

        lax.fori_loop(0, _NSB // 2, body, 0)

        for b in (0, 1):
            gsb = base_sb + _NSB - 2 + b
            for r in range(4):
                write_copy(gsb, b, r).wait()

    return gather_kernel


_gather = _make_gather()


def kernel(x, table):
    idx = x.T.reshape(_TOTAL)
    m = _gather(idx, table)
    return m.transpose(2, 4, 0, 1, 3).reshape(_BATCH, _N_FIELDS, _DIM)


# speedup vs baseline: 1.2650x; 1.1352x over previous
import functools

import jax
import jax.numpy as jnp
from jax import lax
from jax.experimental import pallas as pl
from jax.experimental.pallas import tpu as pltpu
from jax.experimental.pallas import tpu_sc as plsc

_NC = 2
_NS = 16
_NW = _NC * _NS

_BATCH = 16384
_N_FIELDS = 26
_DIM = 32
_VOCAB = 1000000
_TOTAL = _BATCH * _N_FIELDS
_BLK = 128
_CBLK = _BATCH // _BLK
_SB = 4
_SB_ROWS = _SB * _BLK
_NSB = _TOTAL // (_SB_ROWS * _NW)
_IDX_PER_W = _NSB * _SB_ROWS
_SB_PER_F = _CBLK // _SB

_FULL_COLS = _VOCAB // _BLK
_COLS_PER_W = _FULL_COLS // _NW
_COL_REM = _FULL_COLS - _COLS_PER_W * _NW
_TAIL_ROWS = _VOCAB - _FULL_COLS * _BLK
_ROWS2 = _VOCAB * _DIM // 128


def _make_convert():
    mesh = plsc.VectorSubcoreMesh(core_axis_name="c", subcore_axis_name="s")

    @functools.partial(
        pl.kernel,
        mesh=mesh,
        compiler_params=pltpu.CompilerParams(use_tc_tiling_on_sc=True,
                                             needs_layout_passes=False),
        out_type=jax.ShapeDtypeStruct((_ROWS2, 128), jnp.float32),
        scratch_types=[
            pltpu.VMEM((4, 8, 128), jnp.float32),
            pltpu.VMEM((4, 8, 128), jnp.float32),
            pltpu.VMEM((32, 128), jnp.float32),
            pltpu.VMEM((32, 128), jnp.float32),
            pltpu.SemaphoreType.DMA,
            pltpu.SemaphoreType.DMA,
            pltpu.SemaphoreType.DMA,
            pltpu.SemaphoreType.DMA,
        ],
    )
    def convert_kernel(tview_hbm, tail_hbm, rows2_hbm,
                       bin0, bin1, obuf0, obuf1, i0, i1, o0, o1):
        wid = lax.axis_index("s") * _NC + lax.axis_index("c")
        lo = _COLS_PER_W * wid + jnp.minimum(wid, _COL_REM)
        n_cols = _COLS_PER_W + jnp.where(wid < _COL_REM, 1, 0)

        bins = (bin0, bin1)
        obufs = (obuf0, obuf1)
        isem = (i0, i1)
        osem = (o0, o1)

        lane = lax.broadcasted_iota(jnp.int32, (16,), 0)
        s_idx = lax.rem(lane, 8)
        r_even = lane // 8
        r_odd = lane // 8 + 2
        r_idx = (r_even, r_odd)

        @pl.when(wid == 0)
        def _tail():
            pltpu.sync_copy(tail_hbm, obuf0.at[pl.ds(0, 16)])
            pltpu.sync_copy(obuf0.at[pl.ds(0, 16)],
                            rows2_hbm.at[pl.ds(_ROWS2 - 16, 16)])

        def in_copy(c, b):
            return pltpu.make_async_copy(
                tview_hbm.at[:, :, pl.ds(c * _BLK, _BLK)], bins[b], isem[b])

        def out_copy(c, b):
            return pltpu.make_async_copy(
                obufs[b], rows2_hbm.at[pl.ds(32 * c, 32)], osem[b])

        for b in (0, 1):
            @pl.when(n_cols > b)
            def _prime():
                in_copy(lo + b, b).start()

        def body(i, carry):
            for b in (0, 1):
                j = 2 * i + b

                @pl.when(j < n_cols)
                def _process():
                    c = lo + j

                    @pl.when(i >= 1)
                    def _wait_out():
                        out_copy(c, b).wait()

                    in_copy(c, b).wait()

                    bin_b = bins[b]
                    obuf_b = obufs[b]

                    @plsc.parallel_loop(0, 32, unroll=2)
                    def _transpose(q):
                        for e in range(8):
                            l_idx = jnp.full((16,), 4 * q + e // 2, jnp.int32)
                            vec = plsc.load_gather(
                                bin_b, [r_idx[e % 2], s_idx, l_idx])
                            obuf_b[q, pl.ds(16 * e, 16)] = vec

                    out_copy(c, b).start()

                    @pl.when(j + 2 < n_cols)
                    def _next_in():
                        in_copy(c + 2, b).start()
            return carry

        lax.fori_loop(0, (_COLS_PER_W + 2) // 2, body, 0)

        for b in (0, 1):
            @pl.when(n_cols > b)
            def _drain():
                out_copy(lo, b).wait()

    return convert_kernel


def _make_gather():
    mesh = plsc.VectorSubcoreMesh(core_axis_name="c", subcore_axis_name="s")

    @functools.partial(
        pl.kernel,
        mesh=mesh,
        compiler_params=pltpu.CompilerParams(use_tc_tiling_on_sc=False,
                                             needs_layout_passes=False),
        out_type=jax.ShapeDtypeStruct((_N_FIELDS, 4, _CBLK, 8, 128),
                                      jnp.float32),
        scratch_types=[
            pltpu.VMEM((_IDX_PER_W,), jnp.int32),
            pltpu.VMEM((_SB_ROWS, _DIM), jnp.float32),
            pltpu.VMEM((_SB_ROWS, _DIM), jnp.float32),
            pltpu.VMEM((4, _SB, 8, 128), jnp.float32),
            pltpu.VMEM((4, _SB, 8, 128), jnp.float32),
            pltpu.SemaphoreType.DMA,
            pltpu.SemaphoreType.DMA,
            pltpu.SemaphoreType.DMA,
            pltpu.SemaphoreType.DMA,
        ],
    )
    def gather_kernel(idx_hbm, table_hbm, out_hbm,
                      idx_v, rows0, rows1, tiles0, tiles1, g0, g1, w0, w1):
        wid = lax.axis_index("s") * _NC + lax.axis_index("c")
        base_sb = wid * _NSB
        pltpu.sync_copy(idx_hbm.at[pl.ds(wid * _IDX_PER_W, _IDX_PER_W)],
                        idx_v)

        rows = (rows0, rows1)
        tiles = (tiles0, tiles1)
        gsem = (g0, g1)
        wsem = (w0, w1)
        lane = lax.broadcasted_iota(jnp.int32, (16,), 0)
        rvec = [[128 * cc + 16 * e + lane for e in range(8)]
                for cc in range(_SB)]

        def gather_copy(jsb, b):
            return pltpu.make_async_copy(
                table_hbm.at[idx_v.at[pl.ds(jsb * _SB_ROWS, _SB_ROWS)]],
                rows[b], gsem[b])

        def write_copy(gsb, b, r):
            f = gsb // _SB_PER_F
            c0 = lax.rem(gsb, _SB_PER_F) * _SB
            return pltpu.make_async_copy(
                tiles[b].at[r], out_hbm.at[f, r, pl.ds(c0, _SB)], wsem[b])

        for b in (0, 1):
            gather_copy(b, b).start()

        def body(i, carry):
            for b in (0, 1):
                jsb = 2 * i + b
                gsb = base_sb + jsb

                @pl.when(i >= 1)
                def _wait_writes():
                    for r in range(4):
                        write_copy(gsb, b, r).wait()

                gather_copy(jsb, b).wait()

                rows_b = rows[b]
                tiles_b = tiles[b]

                @plsc.parallel_loop(0, 32, unroll=2)
                def _transpose(j):
                    col = jnp.full((16,), j, jnp.int32)
                    r = j // 8
                    s = lax.rem(j, 8)
                    for cc in range(_SB):
                        for e in range(8):
                            vec = plsc.load_gather(rows_b, [rvec[cc][e], col])
                            tiles_b[r, cc, s, pl.ds(16 * e, 16)] = vec

                for r in range(4):
                    write_copy(gsb, b, r).start()

                @pl.when(i < (_NSB // 2) - 1)
                def _next_gather():
                    gather_copy(jsb + 2, b).start()
            return carry

        lax.fori_loop(0, _NSB // 2, body, 0)

        for b in (0, 1):
            gsb = base_sb + _NSB - 2 + b
            for r in range(4):
                write_copy(gsb, b, r).wait()

    return gather_kernel


_convert = _make_convert()
_gather = _make_gather()


def kernel(x, table):
    tview = table.T.reshape(4, 8, _VOCAB)
    tail = table[_VOCAB - _TAIL_ROWS:].reshape(16, 128)
    rows2 = _convert(tview, tail)
    idx = x.T.reshape(_TOTAL)
    m = _gather(idx, rows2.reshape(_VOCAB, _DIM))
    return m.transpose(2, 4, 0, 1, 3).reshape(_BATCH, _N_FIELDS, _DIM)
